# Initial kernel scaffold; baseline (speedup 1.0000x reference)
import jax
import jax.numpy as jnp
from jax.experimental import pallas as pl


def kernel(x_, edge_index, edge_weight, z, Wt0, bt0, Wt1, bt1, Wc0, bc0, Wc1, bc1, gn_w, gn_b, gn_ms):
    raise NotImplementedError("write your pallas kernel here")


# speedup vs baseline: 10.3444x; 10.3444x over previous
import functools

import jax
import jax.numpy as jnp
from jax import lax
from jax.experimental import pallas as pl
from jax.experimental.pallas import tpu as pltpu
from jax.experimental.pallas import tpu_sc as plsc

_N = 10000
_D = 128
_E = 320000
_ZR = 0.8
_EPS = 1e-5

_NC = 2
_NS = 16
_NW = _NC * _NS
_EPW = _E // _NW
_C = 80
_NCHUNK = _EPW // _C
_NPAD = 10240
_STRIPE = _NPAD // _NS


def _pre_body(x_ref, z_ref, wt0_ref, bt0_ref, wt1_ref, bt1_ref,
              xout_ref, xself_ref):
    x = x_ref[...]
    x0 = jnp.maximum(
        jnp.dot(x, wt0_ref[...], preferred_element_type=jnp.float32)
        + bt0_ref[...], 0.0)
    x1 = jnp.maximum(
        jnp.dot(x, wt1_ref[...], preferred_element_type=jnp.float32)
        + bt1_ref[...], 0.0)
    x_in = _ZR * x1 + (1.0 - _ZR) * x0
    x_out = _ZR * x0 + (1.0 - _ZR) * x1
    mask = z_ref[...] > 0.5
    xout_ref[...] = x_out
    xself_ref[...] = jnp.where(mask, x_in, x_out)


def _post_body(s_ref, degp_ref, z_ref, xself_ref, wc0_ref, bc0_ref,
               wc1_ref, bc1_ref, gnw_ref, gnb_ref, gnms_ref, o_ref):
    s = s_ref[0] + s_ref[1]
    deg = degp_ref[0] + degp_ref[1]
    deg = jnp.where(deg < 0.5, deg + 1.0, deg)
    out = s / deg
    mean = jnp.mean(out, axis=0, keepdims=True)
    o = out - mean * gnms_ref[...]
    var = jnp.mean(o * o, axis=0, keepdims=True)
    outn = gnw_ref[...] * o * lax.rsqrt(var + _EPS) + gnb_ref[...]
    xs = xself_ref[...]
    y0 = (jnp.dot(outn, wc0_ref[0:_D], preferred_element_type=jnp.float32)
          + jnp.dot(xs, wc0_ref[_D:2 * _D], preferred_element_type=jnp.float32)
          + bc0_ref[...])
    y1 = (jnp.dot(outn, wc1_ref[0:_D], preferred_element_type=jnp.float32)
          + jnp.dot(xs, wc1_ref[_D:2 * _D], preferred_element_type=jnp.float32)
          + bc1_ref[...])
    out_in = _ZR * y1 + (1.0 - _ZR) * y0
    out_out = _ZR * y0 + (1.0 - _ZR) * y1
    o_ref[...] = jnp.where(z_ref[...] > 0.5, out_in, out_out)


_sc_mesh = plsc.VectorSubcoreMesh(core_axis_name="c", subcore_axis_name="s")


@functools.partial(
    pl.kernel,
    out_type=(jax.ShapeDtypeStruct((_NC, _NPAD, _D), jnp.float32),
              jax.ShapeDtypeStruct((_NC, _NPAD), jnp.float32)),
    mesh=_sc_mesh,
    scratch_types=[
        pltpu.VMEM((_C,), jnp.int32),
        pltpu.VMEM((_C,), jnp.int32),
        pltpu.VMEM((_C,), jnp.float32),
        pltpu.VMEM((_C, _D), jnp.float32),
        pltpu.VMEM_SHARED((_NPAD, _D), jnp.float32),
        pltpu.VMEM_SHARED((_NPAD,), jnp.float32),
        pltpu.SemaphoreType.DMA,
    ],
)
def _sc_scatter(x_hbm, row_hbm, col_hbm, w_hbm, z2_hbm, z1_hbm,
                s_out, deg_out, row_v, col_v, w_v, rows_v, s_sh, deg_sh, sem):
    c = lax.axis_index("c")
    s = lax.axis_index("s")
    wid = c * _NS + s

    pltpu.sync_copy(z2_hbm.at[pl.ds(s * _STRIPE, _STRIPE)],
                    s_sh.at[pl.ds(s * _STRIPE, _STRIPE)])
    pltpu.sync_copy(z1_hbm.at[pl.ds(s * _STRIPE, _STRIPE)],
                    deg_sh.at[pl.ds(s * _STRIPE, _STRIPE)])
    plsc.subcore_barrier()

    base = wid * _EPW

    def chunk_body(k, carry):
        off = base + k * _C
        pltpu.sync_copy(row_hbm.at[pl.ds(off, _C)], row_v)
        pltpu.sync_copy(col_hbm.at[pl.ds(off, _C)], col_v)
        pltpu.sync_copy(w_hbm.at[pl.ds(off, _C)], w_v)
        pltpu.async_copy(x_hbm.at[col_v], rows_v, sem).wait()

        def g_body(g, carry2):
            w16 = w_v[pl.ds(g * 16, 16)]
            for i in range(16):
                ws = w16[i]
                e = g * 16 + i
                for j in range(_D // 16):
                    sl = pl.ds(j * 16, 16)
                    rows_v[e, sl] = rows_v[e, sl] * ws
            return carry2

        lax.fori_loop(0, _C // 16, g_body, 0, unroll=False)
        pltpu.sync_copy(w_v, deg_sh.at[row_v], add=True)
        pltpu.sync_copy(rows_v, s_sh.at[row_v], add=True)
        return carry

    lax.fori_loop(0, _NCHUNK, chunk_body, 0, unroll=False)
    plsc.subcore_barrier()

    pltpu.sync_copy(s_sh.at[pl.ds(s * _STRIPE, _STRIPE)],
                    s_out.at[c, pl.ds(s * _STRIPE, _STRIPE)])
    pltpu.sync_copy(deg_sh.at[pl.ds(s * _STRIPE, _STRIPE)],
                    deg_out.at[c, pl.ds(s * _STRIPE, _STRIPE)])


def kernel(x_, edge_index, edge_weight, z, Wt0, bt0, Wt1, bt1,
           Wc0, bc0, Wc1, bc1, gn_w, gn_b, gn_ms):
    row = edge_index[0]
    col = edge_index[1]
    z2 = z.reshape(_N, 1)

    x_out, x_self = pl.pallas_call(
        _pre_body,
        out_shape=(jax.ShapeDtypeStruct((_N, _D), jnp.float32),
                   jax.ShapeDtypeStruct((_N, _D), jnp.float32)),
    )(x_, z2, Wt0, bt0.reshape(1, _D), Wt1, bt1.reshape(1, _D))

    zeros2 = jnp.zeros((_NPAD, _D), jnp.float32)
    zeros1 = jnp.zeros((_NPAD,), jnp.float32)
    s_part, deg_part = _sc_scatter(x_out, row, col, edge_weight,
                                   zeros2, zeros1)
    s_part = s_part[:, :_N]
    deg_part = deg_part[:, :_N]

    out = pl.pallas_call(
        _post_body,
        out_shape=jax.ShapeDtypeStruct((_N, _D), jnp.float32),
    )(s_part, deg_part.reshape(_NC, _N, 1), z2, x_self,
      Wc0, bc0.reshape(1, _D), Wc1, bc1.reshape(1, _D),
      gn_w.reshape(1, _D), gn_b.reshape(1, _D), gn_ms.reshape(1, _D))
    return out


# speedup vs baseline: 13.9113x; 1.3448x over previous
import functools

import jax
import jax.numpy as jnp
from jax import lax
from jax.experimental import pallas as pl
from jax.experimental.pallas import tpu as pltpu
from jax.experimental.pallas import tpu_sc as plsc

_N = 10000
_D = 128
_E = 320000
_ZR = 0.8
_EPS = 1e-5

_NC = 2
_NS = 16
_NW = _NC * _NS
_EPW = _E // _NW
_C = 80
_NCHUNK = _EPW // _C
_NPAD = 10240
_STRIPE = _NPAD // _NS


def _pre_body(x_ref, z_ref, wt0_ref, bt0_ref, wt1_ref, bt1_ref,
              xout_ref, xself_ref):
    x = x_ref[...]
    x0 = jnp.maximum(
        jnp.dot(x, wt0_ref[...], preferred_element_type=jnp.float32)
        + bt0_ref[...], 0.0)
    x1 = jnp.maximum(
        jnp.dot(x, wt1_ref[...], preferred_element_type=jnp.float32)
        + bt1_ref[...], 0.0)
    x_in = _ZR * x1 + (1.0 - _ZR) * x0
    x_out = _ZR * x0 + (1.0 - _ZR) * x1
    mask = z_ref[...] > 0.5
    xout_ref[...] = x_out
    xself_ref[...] = jnp.where(mask, x_in, x_out)


def _post_body(s_ref, degp_ref, z_ref, xself_ref, wc0_ref, bc0_ref,
               wc1_ref, bc1_ref, gnw_ref, gnb_ref, gnms_ref, o_ref):
    s = s_ref[0] + s_ref[1]
    deg = degp_ref[0] + degp_ref[1]
    deg = jnp.where(deg < 0.5, deg + 1.0, deg)
    out = s / deg
    mean = jnp.mean(out, axis=0, keepdims=True)
    o = out - mean * gnms_ref[...]
    var = jnp.mean(o * o, axis=0, keepdims=True)
    outn = gnw_ref[...] * o * lax.rsqrt(var + _EPS) + gnb_ref[...]
    xs = xself_ref[...]
    y0 = (jnp.dot(outn, wc0_ref[0:_D], preferred_element_type=jnp.float32)
          + jnp.dot(xs, wc0_ref[_D:2 * _D], preferred_element_type=jnp.float32)
          + bc0_ref[...])
    y1 = (jnp.dot(outn, wc1_ref[0:_D], preferred_element_type=jnp.float32)
          + jnp.dot(xs, wc1_ref[_D:2 * _D], preferred_element_type=jnp.float32)
          + bc1_ref[...])
    out_in = _ZR * y1 + (1.0 - _ZR) * y0
    out_out = _ZR * y0 + (1.0 - _ZR) * y1
    o_ref[...] = jnp.where(z_ref[...] > 0.5, out_in, out_out)


_sc_mesh = plsc.VectorSubcoreMesh(core_axis_name="c", subcore_axis_name="s")


@functools.partial(
    pl.kernel,
    out_type=(jax.ShapeDtypeStruct((_NC, _NPAD, _D), jnp.float32),
              jax.ShapeDtypeStruct((_NC, _NPAD), jnp.float32)),
    mesh=_sc_mesh,
    scratch_types=[
        pltpu.VMEM((2, 2, _C), jnp.int32),
        pltpu.VMEM((2, _C), jnp.float32),
        pltpu.VMEM((2, _C, _D), jnp.float32),
        pltpu.VMEM_SHARED((_NPAD, _D), jnp.float32),
        pltpu.VMEM_SHARED((_NPAD,), jnp.float32),
        pltpu.SemaphoreType.DMA((2,)),
        pltpu.SemaphoreType.DMA((2,)),
    ],
)
def _sc_scatter(x_hbm, packed_hbm, w_hbm, z2_hbm, z1_hbm,
                s_out, deg_out, pk_v, wf_v, rows_v, s_sh, deg_sh,
                gsem, ssem):
    c = lax.axis_index("c")
    s = lax.axis_index("s")
    wid = c * _NS + s

    pltpu.sync_copy(z2_hbm.at[pl.ds(s * _STRIPE, _STRIPE)],
                    s_sh.at[pl.ds(s * _STRIPE, _STRIPE)])
    pltpu.sync_copy(z1_hbm.at[pl.ds(s * _STRIPE, _STRIPE)],
                    deg_sh.at[pl.ds(s * _STRIPE, _STRIPE)])
    plsc.subcore_barrier()

    def load_chunk(k, b):
        pltpu.sync_copy(packed_hbm.at[wid, k], pk_v.at[b])
        pltpu.sync_copy(w_hbm.at[wid, k], wf_v.at[b])
        pltpu.async_copy(x_hbm.at[pk_v.at[b, 1]], rows_v.at[b],
                         gsem.at[b])

    def scat_descs(b):
        return (pltpu.make_async_copy(wf_v.at[b],
                                      deg_sh.at[pk_v.at[b, 0]],
                                      ssem.at[b]),
                pltpu.make_async_copy(rows_v.at[b],
                                      s_sh.at[pk_v.at[b, 0]],
                                      ssem.at[b]))

    def scale(b):
        def g_body(g, carry2):
            w16 = wf_v[b, pl.ds(g * 16, 16)]
            for i in range(16):
                ws = w16[i]
                e = g * 16 + i
                for j in range(_D // 16):
                    sl = pl.ds(j * 16, 16)
                    rows_v[b, e, sl] = rows_v[b, e, sl] * ws
            return carry2

        lax.fori_loop(0, _C // 16, g_body, 0, unroll=False)

    def chunk_step(k, b):
        pltpu.make_async_copy(x_hbm.at[pk_v.at[b, 1]], rows_v.at[b],
                              gsem.at[b]).wait()
        b2 = 1 - b

        @pl.when(jnp.logical_and(k + 1 < _NCHUNK, k >= 1))
        def _refill():
            d1, d2 = scat_descs(b2)
            d1.wait()
            d2.wait()
            load_chunk(k + 1, b2)

        @pl.when(jnp.logical_and(k + 1 < _NCHUNK, k < 1))
        def _refill_first():
            load_chunk(k + 1, b2)

        scale(b)
        pltpu.async_copy(wf_v.at[b], deg_sh.at[pk_v.at[b, 0]],
                         ssem.at[b], add=True)
        pltpu.async_copy(rows_v.at[b], s_sh.at[pk_v.at[b, 0]],
                         ssem.at[b], add=True)

    load_chunk(0, 0)

    def pair_body(m, carry):
        chunk_step(2 * m, 0)
        chunk_step(2 * m + 1, 1)
        return carry

    lax.fori_loop(0, _NCHUNK // 2, pair_body, 0, unroll=False)
    chunk_step(_NCHUNK - 1, 0)

    for b in (1, 0):
        d1, d2 = scat_descs(b)
        d1.wait()
        d2.wait()
    plsc.subcore_barrier()

    pltpu.sync_copy(s_sh.at[pl.ds(s * _STRIPE, _STRIPE)],
                    s_out.at[c, pl.ds(s * _STRIPE, _STRIPE)])
    pltpu.sync_copy(deg_sh.at[pl.ds(s * _STRIPE, _STRIPE)],
                    deg_out.at[c, pl.ds(s * _STRIPE, _STRIPE)])


def kernel(x_, edge_index, edge_weight, z, Wt0, bt0, Wt1, bt1,
           Wc0, bc0, Wc1, bc1, gn_w, gn_b, gn_ms):
    row = edge_index[0]
    col = edge_index[1]
    z2 = z.reshape(_N, 1)

    x_out, x_self = pl.pallas_call(
        _pre_body,
        out_shape=(jax.ShapeDtypeStruct((_N, _D), jnp.float32),
                   jax.ShapeDtypeStruct((_N, _D), jnp.float32)),
    )(x_, z2, Wt0, bt0.reshape(1, _D), Wt1, bt1.reshape(1, _D))

    zeros2 = jnp.zeros((_NPAD, _D), jnp.float32)
    zeros1 = jnp.zeros((_NPAD,), jnp.float32)
    row3 = row.reshape(_NW, _NCHUNK, _C)
    col3 = col.reshape(_NW, _NCHUNK, _C)
    w3 = edge_weight.reshape(_NW, _NCHUNK, _C)
    packed = jnp.stack([row3, col3], axis=2)
    s_part, deg_part = _sc_scatter(x_out, packed, w3, zeros2, zeros1)
    s_part = s_part[:, :_N]
    deg_part = deg_part[:, :_N]

    out = pl.pallas_call(
        _post_body,
        out_shape=jax.ShapeDtypeStruct((_N, _D), jnp.float32),
    )(s_part, deg_part.reshape(_NC, _N, 1), z2, x_self,
      Wc0, bc0.reshape(1, _D), Wc1, bc1.reshape(1, _D),
      gn_w.reshape(1, _D), gn_b.reshape(1, _D), gn_ms.reshape(1, _D))
    return out


# speedup vs baseline: 21.3875x; 1.5374x over previous
import functools

import jax
import jax.numpy as jnp
from jax import lax
from jax.experimental import pallas as pl
from jax.experimental.pallas import tpu as pltpu
from jax.experimental.pallas import tpu_sc as plsc

_N = 10000
_D = 128
_E = 320000
_ZR = 0.8
_EPS = 1e-5

_NC = 2
_NS = 16
_NW = _NC * _NS
_EPW = _E // _NW
_C = 80
_NCHUNK = _EPW // _C
_NPAD = 10240
_STRIPE = _NPAD // _NS


def _pre_body(x_ref, z_ref, wt0_ref, bt0_ref, wt1_ref, bt1_ref,
              xout_ref, xself_ref):
    x = x_ref[...]
    x0 = jnp.maximum(
        jnp.dot(x, wt0_ref[...], preferred_element_type=jnp.float32)
        + bt0_ref[...], 0.0)
    x1 = jnp.maximum(
        jnp.dot(x, wt1_ref[...], preferred_element_type=jnp.float32)
        + bt1_ref[...], 0.0)
    x_in = _ZR * x1 + (1.0 - _ZR) * x0
    x_out = _ZR * x0 + (1.0 - _ZR) * x1
    mask = z_ref[...] > 0.5
    xout_ref[...] = x_out
    xself_ref[...] = jnp.where(mask, x_in, x_out)


def _post_body(s_ref, degp_ref, z_ref, xself_ref, wc0_ref, bc0_ref,
               wc1_ref, bc1_ref, gnw_ref, gnb_ref, gnms_ref, o_ref):
    s = s_ref[0, 0:_N] + s_ref[1, 0:_N]
    deg = degp_ref[0, 0:_N] + degp_ref[1, 0:_N]
    deg = jnp.where(deg < 0.5, deg + 1.0, deg)
    out = s / deg
    mean = jnp.mean(out, axis=0, keepdims=True)
    o = out - mean * gnms_ref[...]
    var = jnp.mean(o * o, axis=0, keepdims=True)
    outn = gnw_ref[...] * o * lax.rsqrt(var + _EPS) + gnb_ref[...]
    xs = xself_ref[...]
    y0 = (jnp.dot(outn, wc0_ref[0:_D], preferred_element_type=jnp.float32)
          + jnp.dot(xs, wc0_ref[_D:2 * _D], preferred_element_type=jnp.float32)
          + bc0_ref[...])
    y1 = (jnp.dot(outn, wc1_ref[0:_D], preferred_element_type=jnp.float32)
          + jnp.dot(xs, wc1_ref[_D:2 * _D], preferred_element_type=jnp.float32)
          + bc1_ref[...])
    out_in = _ZR * y1 + (1.0 - _ZR) * y0
    out_out = _ZR * y0 + (1.0 - _ZR) * y1
    o_ref[...] = jnp.where(z_ref[...] > 0.5, out_in, out_out)


_sc_mesh = plsc.VectorSubcoreMesh(core_axis_name="c", subcore_axis_name="s")


@functools.partial(
    pl.kernel,
    out_type=(jax.ShapeDtypeStruct((_NC, _NPAD, _D), jnp.float32),
              jax.ShapeDtypeStruct((_NC, _NPAD), jnp.float32)),
    mesh=_sc_mesh,
    scratch_types=[
        pltpu.VMEM((_NCHUNK, _C), jnp.int32),
        pltpu.VMEM((2, _C), jnp.float32),
        pltpu.VMEM((2, _C), jnp.int32),
        pltpu.VMEM((2, _C), jnp.int32),
        pltpu.VMEM((2, _C, _D), jnp.float32),
        pltpu.VMEM_SHARED((_NPAD, _D), jnp.float32),
        pltpu.VMEM_SHARED((_NPAD,), jnp.float32),
        pltpu.SemaphoreType.DMA((2,)),
        pltpu.SemaphoreType.DMA((2,)),
    ],
)
def _sc_scatter(x_hbm, rc_hbm, w_hbm, z2_hbm, z1_hbm,
                s_out, deg_out, rc_v, wf_v, row_v, col_v, rows_v,
                s_sh, deg_sh, gsem, ssem):
    c = lax.axis_index("c")
    s = lax.axis_index("s")
    wid = c * _NS + s
    w_hbm = w_hbm.at[wid]

    pltpu.sync_copy(rc_hbm.at[wid], rc_v)

    pltpu.sync_copy(z2_hbm.at[pl.ds(s * _STRIPE, _STRIPE)],
                    s_sh.at[pl.ds(s * _STRIPE, _STRIPE)])
    pltpu.sync_copy(z1_hbm.at[pl.ds(s * _STRIPE, _STRIPE)],
                    deg_sh.at[pl.ds(s * _STRIPE, _STRIPE)])
    plsc.subcore_barrier()

    def start_chunk(k, b):
        for g in range(_C // 16):
            sl = pl.ds(g * 16, 16)
            rc16 = rc_v[k, sl]
            row_v[b, sl] = lax.shift_right_logical(rc16, 14)
            col_v[b, sl] = jnp.bitwise_and(rc16, 16383)
        pltpu.async_copy(w_hbm.at[k], wf_v.at[b], gsem.at[b])
        pltpu.async_copy(x_hbm.at[col_v.at[b]], rows_v.at[b],
                         gsem.at[b])

    def scat_descs(b):
        return (pltpu.make_async_copy(wf_v.at[b],
                                      deg_sh.at[row_v.at[b]],
                                      ssem.at[b]),
                pltpu.make_async_copy(rows_v.at[b],
                                      s_sh.at[row_v.at[b]],
                                      ssem.at[b]))

    def scale(b):
        for g in range(_C // 16):
            w16 = wf_v[b, pl.ds(g * 16, 16)]
            for i in range(16):
                ws = w16[i]
                e = g * 16 + i
                for j in range(_D // 16):
                    sl = pl.ds(j * 16, 16)
                    rows_v[b, e, sl] = rows_v[b, e, sl] * ws

    def chunk_step(k, b):
        pltpu.make_async_copy(w_hbm.at[0], wf_v.at[b], gsem.at[b]).wait()
        pltpu.make_async_copy(x_hbm.at[col_v.at[b]], rows_v.at[b],
                              gsem.at[b]).wait()
        b2 = 1 - b

        @pl.when(jnp.logical_and(k + 1 < _NCHUNK, k >= 1))
        def _refill():
            d1, d2 = scat_descs(b2)
            d1.wait()
            d2.wait()
            start_chunk(k + 1, b2)

        @pl.when(jnp.logical_and(k + 1 < _NCHUNK, k < 1))
        def _refill_first():
            start_chunk(k + 1, b2)

        scale(b)
        pltpu.async_copy(wf_v.at[b], deg_sh.at[row_v.at[b]],
                         ssem.at[b], add=True)
        pltpu.async_copy(rows_v.at[b], s_sh.at[row_v.at[b]],
                         ssem.at[b], add=True)

    start_chunk(0, 0)

    def pair_body(m, carry):
        chunk_step(2 * m, 0)
        chunk_step(2 * m + 1, 1)
        return carry

    lax.fori_loop(0, _NCHUNK // 2, pair_body, 0, unroll=False)
    chunk_step(_NCHUNK - 1, 0)

    for b in (1, 0):
        d1, d2 = scat_descs(b)
        d1.wait()
        d2.wait()
    plsc.subcore_barrier()

    pltpu.sync_copy(s_sh.at[pl.ds(s * _STRIPE, _STRIPE)],
                    s_out.at[c, pl.ds(s * _STRIPE, _STRIPE)])
    pltpu.sync_copy(deg_sh.at[pl.ds(s * _STRIPE, _STRIPE)],
                    deg_out.at[c, pl.ds(s * _STRIPE, _STRIPE)])


def kernel(x_, edge_index, edge_weight, z, Wt0, bt0, Wt1, bt1,
           Wc0, bc0, Wc1, bc1, gn_w, gn_b, gn_ms):
    row = edge_index[0]
    col = edge_index[1]
    z2 = z.reshape(_N, 1)

    x_out, x_self = pl.pallas_call(
        _pre_body,
        out_shape=(jax.ShapeDtypeStruct((_N, _D), jnp.float32),
                   jax.ShapeDtypeStruct((_N, _D), jnp.float32)),
    )(x_, z2, Wt0, bt0.reshape(1, _D), Wt1, bt1.reshape(1, _D))

    zeros2 = jnp.zeros((_NPAD, _D), jnp.float32)
    zeros1 = jnp.zeros((_NPAD,), jnp.float32)
    rc = (row * 16384 + col).reshape(_NW, _NCHUNK, _C)
    w3 = edge_weight.reshape(_NW, _NCHUNK, _C)
    s_part, deg_part = _sc_scatter(x_out, rc, w3, zeros2, zeros1)

    out = pl.pallas_call(
        _post_body,
        out_shape=jax.ShapeDtypeStruct((_N, _D), jnp.float32),
    )(s_part, deg_part.reshape(_NC, _NPAD, 1), z2, x_self,
      Wc0, bc0.reshape(1, _D), Wc1, bc1.reshape(1, _D),
      gn_w.reshape(1, _D), gn_b.reshape(1, _D), gn_ms.reshape(1, _D))
    return out
